# baseline (device time: 25684 ns/iter reference)
import jax
import jax.numpy as jnp
from jax import lax
from jax.experimental import pallas as pl
from jax.experimental.pallas import tpu as pltpu

N_Y = 4
V_CHUNK = 2048


def kernel(x, W, labels):
    T, D = x.shape
    V = W.shape[1]
    n_chunks = V // V_CHUNK

    def body(x_ref, w_ref, lab_ref, out_ref,
             acc_ref, comm_ref, send_sems, recv_sems):
        i = pl.program_id(0)
        my_x = lax.axis_index("x")
        my_y = lax.axis_index("y")
        my_z = lax.axis_index("z")

        barrier = pltpu.get_barrier_semaphore()

        @pl.when(i == 0)
        def _signal_peers():
            for d in range(1, N_Y):
                pl.semaphore_signal(
                    barrier, inc=1,
                    device_id=(my_x, (my_y + d) % N_Y, my_z),
                    device_id_type=pl.DeviceIdType.MESH,
                )

        logits = jnp.dot(x_ref[...], w_ref[...],
                         preferred_element_type=jnp.float32)
        cs = jnp.sum(jnp.exp(logits), axis=1)
        cols = (lax.broadcasted_iota(jnp.int32, (T, V_CHUNK), 1)
                + my_y * V + i * V_CHUNK)
        hit = cols == lab_ref[...][:, None]
        cg = jnp.sum(jnp.where(hit, logits, 0.0), axis=1)

        @pl.when(i == 0)
        def _init():
            acc_ref[0] = cs
            acc_ref[1] = cg

        @pl.when(i > 0)
        def _merge():
            acc_ref[0] = acc_ref[0] + cs
            acc_ref[1] = acc_ref[1] + cg

        @pl.when(i == n_chunks - 1)
        def _exchange_and_combine():
            pl.semaphore_wait(barrier, N_Y - 1)

            comm_ref[0] = acc_ref[...]

            sends = []
            for d in range(1, N_Y):
                rdma = pltpu.make_async_remote_copy(
                    src_ref=comm_ref.at[0],
                    dst_ref=comm_ref.at[d],
                    send_sem=send_sems.at[d - 1],
                    recv_sem=recv_sems.at[d - 1],
                    device_id=(my_x, (my_y + d) % N_Y, my_z),
                    device_id_type=pl.DeviceIdType.MESH,
                )
                rdma.start()
                sends.append(rdma)
            for rdma in sends:
                rdma.wait_recv()
            for rdma in sends:
                rdma.wait_send()

            S = jnp.sum(comm_ref[:, 0, :], axis=0)
            G = jnp.sum(comm_ref[:, 1, :], axis=0)
            out_ref[...] = jnp.log(S) - G

    return pl.pallas_call(
        body,
        grid=(n_chunks,),
        out_shape=jax.ShapeDtypeStruct((T,), jnp.float32),
        in_specs=[
            pl.BlockSpec((T, D), lambda i: (0, 0), memory_space=pltpu.VMEM),
            pl.BlockSpec((D, V_CHUNK), lambda i: (0, i),
                         memory_space=pltpu.VMEM),
            pl.BlockSpec((T,), lambda i: (0,), memory_space=pltpu.VMEM),
        ],
        out_specs=pl.BlockSpec((T,), lambda i: (0,), memory_space=pltpu.VMEM),
        scratch_shapes=[
            pltpu.VMEM((2, T), jnp.float32),
            pltpu.VMEM((N_Y, 2, T), jnp.float32),
            pltpu.SemaphoreType.DMA((N_Y - 1,)),
            pltpu.SemaphoreType.DMA((N_Y - 1,)),
        ],
        compiler_params=pltpu.CompilerParams(collective_id=0),
    )(x, W, labels)
